# baseline (speedup 1.0000x reference)
import jax
import jax.numpy as jnp
import numpy as _np
from jax.experimental import pallas as pl
from jax.experimental.pallas import tpu as pltpu

K = 10
_CHUNK = 2048


def _analysis_kernel(l_ref, idx_ref, best_ref, f_ref):
    l = l_ref[...]
    Bq, N = l.shape
    c = jnp.abs(l)
    iota = jax.lax.broadcasted_iota(jnp.int32, (Bq, N), 1)
    inf = jnp.float32(jnp.inf)
    idx_cols, f_cols, cvals = [], [], []
    cw = c
    for _ in range(K):
        m = jnp.min(cw, axis=1, keepdims=True)
        idx = jnp.min(jnp.where(cw == m, iota, N), axis=1, keepdims=True)
        hit = iota == idx
        sl = jnp.sum(jnp.where(hit, l, 0.0), axis=1, keepdims=True)
        idx_cols.append(idx)
        cvals.append(m)
        f_cols.append(jnp.where(sl > 0.0, -1.0, 1.0).astype(jnp.float32))
        cw = jnp.where(hit, inf, cw)
    M = 1 << K
    miota = jax.lax.broadcasted_iota(jnp.int32, (Bq, M), 1)
    sums = jnp.zeros((Bq, M), jnp.float32)
    for t in range(K):
        bit = ((miota >> t) & 1).astype(jnp.float32)
        cvb = cvals[t].astype(jnp.bfloat16).astype(jnp.float32)
        sums = sums + cvb * bit
    best_cols = []
    sw = sums
    for _ in range(K):
        mj = jnp.min(sw, axis=1, keepdims=True)
        bj = jnp.min(jnp.where(sw == mj, miota, M), axis=1, keepdims=True)
        best_cols.append(bj)
        sw = jnp.where(miota == bj, inf, sw)
    idx_ref[...] = jnp.concatenate(idx_cols, axis=1)
    best_ref[...] = jnp.concatenate(best_cols, axis=1)
    f_ref[...] = jnp.concatenate(f_cols, axis=1)


def _build_kernel(idx_sm, best_sm, f_ref, l_ref, out_ref, distr_ref, ent_ref):
    b = pl.program_id(0)
    Bq = pl.num_programs(0)
    N = l_ref.shape[2]
    lrow = l_ref[0]
    z0 = (lrow > 0.0).astype(jnp.float32)
    iota = jax.lax.broadcasted_iota(jnp.int32, (1, N), 1)
    masks = []
    for t in range(K):
        masks.append((iota == idx_sm[b, t]).astype(jnp.float32))
    onehots = jnp.concatenate(masks, axis=0)
    fr = f_ref[0]
    ciota = jax.lax.broadcasted_iota(jnp.int32, (1, K), 1)
    wrows = []
    for j in range(K):
        bits = ((best_sm[b, j] >> ciota) & 1).astype(jnp.float32)
        wrows.append(bits * fr)
    w = jnp.concatenate(wrows, axis=0)
    zmat = z0 + jnp.dot(w, onehots, preferred_element_type=jnp.float32)
    out_ref[0] = zmat
    zl = (zmat * lrow).astype(jnp.bfloat16)
    ones = jnp.ones((_CHUNK, 1), jnp.bfloat16)
    parts = [jnp.dot(zl[:, i * _CHUNK:(i + 1) * _CHUNK], ones,
                     preferred_element_type=jnp.float32)
             for i in range(N // _CHUNK)]
    while len(parts) > 1:
        h = len(parts) // 2
        parts = [parts[i] + parts[i + h] for i in range(h)]
    s = parts[0]
    vals = [s[j:j + 1] for j in range(K)]
    for phase in range(K):
        for a in range(phase % 2, K - 1, 2):
            lo = jnp.minimum(vals[a], vals[a + 1])
            hi = jnp.maximum(vals[a], vals[a + 1])
            vals[a], vals[a + 1] = hi, lo
    css = [vals[0]]
    for j in range(1, K):
        css.append(css[-1] + vals[j])
    ksup = jnp.zeros((1, 1), jnp.float32)
    for j in range(K):
        ksup = ksup + ((1.0 + float(j + 1) * vals[j]) > css[j]).astype(jnp.float32)
    css_k = jnp.zeros((1, 1), jnp.float32)
    recip = jnp.zeros((1, 1), jnp.float32)
    for j in range(K):
        sel = ksup == float(j + 1)
        css_k = css_k + jnp.where(sel, css[j], 0.0)
        recip = recip + jnp.where(sel, jnp.float32(_np.float32(1.0) / _np.float32(j + 1)), 0.0)
    tau = (css_k - 1.0) * recip
    distr = jnp.maximum(s - tau, 0.0)
    distr_ref[0] = distr
    safe = jnp.where(distr > 0.0, distr, 1.0)
    plogp = jnp.where(distr > 0.0, distr * jnp.log(safe), 0.0)
    rowent = jnp.sum(plogp, axis=(0, 1), keepdims=True)

    @pl.when(b == 0)
    def _():
        ent_ref[...] = jnp.zeros((1, 1), jnp.float32)

    ent_ref[...] += rowent


def kernel(logits):
    Bq, N = logits.shape
    idx_small, best, f2 = pl.pallas_call(
        _analysis_kernel,
        out_shape=(
            jax.ShapeDtypeStruct((Bq, K), jnp.int32),
            jax.ShapeDtypeStruct((Bq, K), jnp.int32),
            jax.ShapeDtypeStruct((Bq, K), jnp.float32),
        ),
    )(logits)
    f3 = f2.reshape(Bq, 1, K)
    grid_spec = pltpu.PrefetchScalarGridSpec(
        num_scalar_prefetch=2,
        grid=(Bq,),
        in_specs=[
            pl.BlockSpec((1, 1, K), lambda b, *_: (b, 0, 0)),
            pl.BlockSpec((1, 1, N), lambda b, *_: (b, 0, 0)),
        ],
        out_specs=[
            pl.BlockSpec((1, K, N), lambda b, *_: (b, 0, 0)),
            pl.BlockSpec((1, K, 1), lambda b, *_: (b, 0, 0)),
            pl.BlockSpec((1, 1), lambda b, *_: (0, 0)),
        ],
    )
    sample, distr3, ent_sum = pl.pallas_call(
        _build_kernel,
        grid_spec=grid_spec,
        out_shape=(
            jax.ShapeDtypeStruct((Bq, K, N), jnp.float32),
            jax.ShapeDtypeStruct((Bq, K, 1), jnp.float32),
            jax.ShapeDtypeStruct((1, 1), jnp.float32),
        ),
    )(idx_small, best, f3, logits.reshape(Bq, 1, N))
    distr = distr3.reshape(Bq, K)
    entropy = (-ent_sum / Bq).reshape(())
    return (sample, distr, entropy)
